# baseline (speedup 1.0000x reference)
import jax
import jax.numpy as jnp
from jax.experimental import pallas as pl
from jax.experimental.pallas import tpu as pltpu

N = 5000
NP = 5120
NW = NP // 16
JC = 32
IT = 1024
IOU_THRESHOLD = 0.5


def _nms_kernel(bcol, x1r, y1r, x2r, y2r, sr, out_ref, p_ref, wt_ref, kp_ref):
    w_iota = jax.lax.broadcasted_iota(jnp.int32, (NW, NP), 0)
    j_iota = jax.lax.broadcasted_iota(jnp.int32, (NW, NP), 1)
    pow_row = (jnp.uint32(1) << (jax.lax.broadcasted_iota(jnp.uint32, (1, NP), 1) & 15)
               ).astype(jnp.float32)
    wt_ref[...] = jnp.where((j_iota >> 4) == w_iota, pow_row, 0.0).astype(jnp.bfloat16)

    h2 = jax.lax.broadcasted_iota(jnp.int32, (2, JC), 0)
    jj = jax.lax.broadcasted_iota(jnp.int32, (2, JC), 1)
    pw = jnp.where((jj >> 4) == h2,
                   (jnp.uint32(1) << (jj & 15).astype(jnp.uint32)).astype(jnp.float32),
                   0.0).astype(jnp.bfloat16)
    ii = jax.lax.broadcasted_iota(jnp.int32, (IT, IT // 16), 0)
    hh = jax.lax.broadcasted_iota(jnp.int32, (IT, IT // 16), 1)
    pwt = jnp.where((ii >> 4) == hh,
                    (jnp.uint32(1) << (ii & 15).astype(jnp.uint32)).astype(jnp.float32),
                    0.0).astype(jnp.bfloat16)

    def build_tile(t, _):
        i0 = t * IT
        x1i = x1r[:, pl.ds(i0, IT)]
        y1i = y1r[:, pl.ds(i0, IT)]
        x2i = x2r[:, pl.ds(i0, IT)]
        y2i = y2r[:, pl.ds(i0, IT)]
        si = sr[:, pl.ds(i0, IT)]
        area_i = (x2i - x1i) * (y2i - y1i)
        i_idx = jax.lax.broadcasted_iota(jnp.int32, (1, IT), 1) + i0

        def masks(cc, k):
            row0 = cc * 128 + k * JC
            bj = bcol[pl.ds(row0, JC), :]
            x1j = bj[:, 0:1]
            y1j = bj[:, 1:2]
            x2j = bj[:, 2:3]
            y2j = bj[:, 3:4]
            sj = bj[:, 4:5]
            j_idx = jax.lax.broadcasted_iota(jnp.int32, (JC, 1), 0) + row0
            area_j = (x2j - x1j) * (y2j - y1j)

            xx1 = jnp.maximum(x1j, x1i)
            yy1 = jnp.minimum(y1j, y1i)
            xx2 = jnp.minimum(x2j, x2i)
            yy2 = jnp.maximum(y2j, y2i)
            inter = jnp.abs(xx2 - xx1) * jnp.abs(yy2 - yy1)
            union = area_j + area_i - inter
            iou = inter / union

            before = (sj > si) | ((sj == si) & (j_idx < i_idx))
            return iou > IOU_THRESHOLD, before

        def full_block(cc, _):
            words = []
            for k in range(4):
                ioum, before = masks(cc, k)
                sup = (ioum & before).astype(jnp.bfloat16)
                words.append(jax.lax.dot_general(
                    pw, sup, (((1,), (0,)), ((), ())),
                    preferred_element_type=jnp.float32))
            p_ref[pl.ds(cc * 8, 8), pl.ds(i0, IT)] = (
                jnp.concatenate(words, axis=0).astype(jnp.uint32))
            return 0

        def sym_block(cc, _):
            words = []
            rwords = []
            for k in range(4):
                ioum, before = masks(cc, k)
                sup = (ioum & before).astype(jnp.bfloat16)
                supb = (ioum & ~before).astype(jnp.bfloat16)
                words.append(jax.lax.dot_general(
                    pw, sup, (((1,), (0,)), ((), ())),
                    preferred_element_type=jnp.float32))
                rwords.append(jax.lax.dot_general(
                    supb, pwt, (((1,), (0,)), ((), ())),
                    preferred_element_type=jnp.float32))
            p_ref[pl.ds(cc * 8, 8), pl.ds(i0, IT)] = (
                jnp.concatenate(words, axis=0).astype(jnp.uint32))

            rt = jnp.transpose(jnp.concatenate(rwords, axis=0))
            p_ref[pl.ds(t * (IT // 16), IT // 16), pl.ds(cc * 128, 128)] = (
                rt.astype(jnp.uint32))
            return 0

        jax.lax.fori_loop(8 * t, 8 * t + 8, full_block, 0)
        jax.lax.fori_loop(0, 8 * t, sym_block, 0)
        return 0

    jax.lax.fori_loop(0, NP // IT, build_tile, 0)

    wt = wt_ref[...]

    def hits():
        def step(c, acc):
            m = p_ref[pl.ds(c * 32, 32), :] & kp_ref[pl.ds(c * 32, 32), :]
            m = m[0:16] | m[16:32]
            m = m[0:8] | m[8:16]
            m = m[0:4] | m[4:8]
            m = m[0:2] | m[2:4]
            return acc | m[0:1] | m[1:2]
        return jax.lax.fori_loop(0, NW // 32, step, jnp.zeros((1, NP), jnp.uint32))

    def cond(changed):
        return changed

    def body(_):
        keep_b = (hits() == 0).astype(jnp.bfloat16)
        keep_b8 = jnp.broadcast_to(keep_b, (8, NP))
        kp_f = jax.lax.dot_general(
            wt, keep_b8, (((1,), (1,)), ((), ())),
            preferred_element_type=jnp.float32,
        )
        kp_new = kp_f[:, 0:1].astype(jnp.uint32)
        changed = jnp.any(kp_new != kp_ref[...])
        kp_ref[...] = kp_new
        return changed

    kp_ref[...] = jnp.full((NW, 1), 0xFFFF, dtype=jnp.uint32)
    jax.lax.while_loop(cond, body, True)
    out_ref[...] = (hits() == 0).astype(jnp.int32)


def kernel(boxes, scores):
    bp = jnp.pad(boxes, ((0, NP - N), (0, 0)))
    sp = jnp.pad(scores, (0, NP - N), constant_values=-jnp.inf)
    x1 = bp[:, 0]
    y1 = bp[:, 3]
    x2 = bp[:, 2]
    y2 = bp[:, 1]
    row = lambda v: v.reshape(1, NP)
    bcol = jnp.stack([x1, y1, x2, y2, sp], axis=1)

    out = pl.pallas_call(
        _nms_kernel,
        out_shape=jax.ShapeDtypeStruct((1, NP), jnp.int32),
        scratch_shapes=[
            pltpu.VMEM((NW, NP), jnp.uint32),
            pltpu.VMEM((NW, NP), jnp.bfloat16),
            pltpu.VMEM((NW, 1), jnp.uint32),
        ],
    )(bcol, row(x1), row(y1), row(x2), row(y2), row(sp))
    return out[0, :N]


# speedup vs baseline: 1.1490x; 1.1490x over previous
import jax
import jax.numpy as jnp
from jax.experimental import pallas as pl
from jax.experimental.pallas import tpu as pltpu

N = 5000
NP = 5120
NW = NP // 32
JC = 32
IT = 1024
IOU_THRESHOLD = 0.5


def _pow16(x):
    return (jnp.uint32(1) << (x & 15).astype(jnp.uint32)).astype(jnp.float32)


def _nms_kernel(bcol, x1r, y1r, x2r, y2r, sr, out_ref, p_ref, wt_ref, kp_ref):
    r_iota = jax.lax.broadcasted_iota(jnp.int32, (2 * NW, NP), 0)
    j_iota = jax.lax.broadcasted_iota(jnp.int32, (2 * NW, NP), 1)
    row_for_j = (j_iota >> 5) + NW * ((j_iota >> 4) & 1)
    pow_row = _pow16(jax.lax.broadcasted_iota(jnp.int32, (1, NP), 1))
    wt_ref[...] = jnp.where(row_for_j == r_iota, pow_row, 0.0).astype(jnp.bfloat16)

    r16 = jax.lax.broadcasted_iota(jnp.int32, (16, 256), 0)
    jj = jax.lax.broadcasted_iota(jnp.int32, (16, 256), 1)
    pw = jnp.where(((jj >> 5) + 8 * ((jj >> 4) & 1)) == r16,
                   _pow16(jj), 0.0).astype(jnp.bfloat16)

    ii = jax.lax.broadcasted_iota(jnp.int32, (IT, IT // 16), 0)
    hh = jax.lax.broadcasted_iota(jnp.int32, (IT, IT // 16), 1)
    pwt = jnp.where(((ii >> 5) + (IT // 32) * ((ii >> 4) & 1)) == hh,
                    _pow16(ii), 0.0).astype(jnp.bfloat16)

    def build_tile(t, _):
        i0 = t * IT
        x1i = x1r[:, pl.ds(i0, IT)]
        y1i = y1r[:, pl.ds(i0, IT)]
        x2i = x2r[:, pl.ds(i0, IT)]
        y2i = y2r[:, pl.ds(i0, IT)]
        si = sr[:, pl.ds(i0, IT)]
        area_i = (x2i - x1i) * (y2i - y1i)
        i_idx = jax.lax.broadcasted_iota(jnp.int32, (1, IT), 1) + i0

        def masks(cc, k):
            row0 = cc * 128 + k * JC
            bj = bcol[pl.ds(row0, JC), :]
            x1j = bj[:, 0:1]
            y1j = bj[:, 1:2]
            x2j = bj[:, 2:3]
            y2j = bj[:, 3:4]
            sj = bj[:, 4:5]
            j_idx = jax.lax.broadcasted_iota(jnp.int32, (JC, 1), 0) + row0
            area_j = (x2j - x1j) * (y2j - y1j)

            xx1 = jnp.maximum(x1j, x1i)
            yy1 = jnp.minimum(y1j, y1i)
            xx2 = jnp.minimum(x2j, x2i)
            yy2 = jnp.maximum(y2j, y2i)
            inter = jnp.abs(xx2 - xx1) * jnp.abs(yy2 - yy1)
            union = area_j + area_i - inter
            iou = inter / union

            before = (sj > si) | ((sj == si) & (j_idx < i_idx))
            return iou > IOU_THRESHOLD, before

        def store_fwd(pp, sups):
            sup256 = jnp.concatenate(sups, axis=0)
            halves = jax.lax.dot_general(
                pw, sup256, (((1,), (0,)), ((), ())),
                preferred_element_type=jnp.float32)
            hu = halves.astype(jnp.uint32)
            p_ref[pl.ds(pp * 8, 8), pl.ds(i0, IT)] = hu[0:8] | (hu[8:16] << 16)

        def bwd_store(cc, bwd):
            sup128b = jnp.concatenate(bwd, axis=0)
            rhalves = jax.lax.dot_general(
                sup128b, pwt, (((1,), (0,)), ((), ())),
                preferred_element_type=jnp.float32)
            rt = jnp.transpose(rhalves)
            rtu = rt.astype(jnp.uint32)
            rw32 = rtu[0:IT // 32] | (rtu[IT // 32:IT // 16] << 16)
            p_ref[pl.ds(t * (IT // 32), IT // 32), pl.ds(cc * 128, 128)] = rw32

        def sym_pair(pp, _):
            sups = []
            for cc in (2 * pp, 2 * pp + 1):
                bwd = []
                for k in range(4):
                    ioum, before = masks(cc, k)
                    sups.append((ioum & before).astype(jnp.bfloat16))
                    bwd.append((ioum & ~before).astype(jnp.bfloat16))
                bwd_store(cc, bwd)
            store_fwd(pp, sups)
            return 0

        def full_pair(pp, _):
            sups = []
            for cc in (2 * pp, 2 * pp + 1):
                for k in range(4):
                    ioum, before = masks(cc, k)
                    sups.append((ioum & before).astype(jnp.bfloat16))
            store_fwd(pp, sups)
            return 0

        jax.lax.fori_loop(0, 4 * t, sym_pair, 0)
        jax.lax.fori_loop(4 * t, 4 * t + 4, full_pair, 0)
        return 0

    jax.lax.fori_loop(0, NP // IT, build_tile, 0)

    wt = wt_ref[...]

    def hits():
        def step(c, acc):
            m = p_ref[pl.ds(c * 32, 32), :] & kp_ref[pl.ds(c * 32, 32), :]
            m = m[0:16] | m[16:32]
            m = m[0:8] | m[8:16]
            m = m[0:4] | m[4:8]
            m = m[0:2] | m[2:4]
            return acc | m[0:1] | m[1:2]
        return jax.lax.fori_loop(0, NW // 32, step, jnp.zeros((1, NP), jnp.uint32))

    def update():
        keep_b = (hits() == 0).astype(jnp.bfloat16)
        keep_b8 = jnp.broadcast_to(keep_b, (8, NP))
        halves = jax.lax.dot_general(
            wt, keep_b8, (((1,), (1,)), ((), ())),
            preferred_element_type=jnp.float32,
        )[:, 0:1].astype(jnp.uint32)
        kp_new = halves[0:NW] | (halves[NW:2 * NW] << 16)
        changed = jnp.any(kp_new != kp_ref[...])
        kp_ref[...] = kp_new
        return changed

    def body(_):
        update()
        return update()

    kp_ref[...] = jnp.full((NW, 1), 0xFFFFFFFF, dtype=jnp.uint32)
    jax.lax.while_loop(lambda changed: changed, body, True)
    out_ref[...] = (hits() == 0).astype(jnp.int32)


def kernel(boxes, scores):
    bp = jnp.pad(boxes, ((0, NP - N), (0, 0)))
    sp = jnp.pad(scores, (0, NP - N), constant_values=-jnp.inf)
    x1 = bp[:, 0]
    y1 = bp[:, 3]
    x2 = bp[:, 2]
    y2 = bp[:, 1]
    row = lambda v: v.reshape(1, NP)
    bcol = jnp.stack([x1, y1, x2, y2, sp], axis=1)

    out = pl.pallas_call(
        _nms_kernel,
        out_shape=jax.ShapeDtypeStruct((1, NP), jnp.int32),
        scratch_shapes=[
            pltpu.VMEM((NW, NP), jnp.uint32),
            pltpu.VMEM((2 * NW, NP), jnp.bfloat16),
            pltpu.VMEM((NW, 1), jnp.uint32),
        ],
    )(bcol, row(x1), row(y1), row(x2), row(y2), row(sp))
    return out[0, :N]
